# baseline (speedup 1.0000x reference)
import functools
import numpy as np
import jax
import jax.numpy as jnp
from jax import lax
from jax.experimental import pallas as pl
from jax.experimental.pallas import tpu as pltpu
from jax.experimental.pallas import tpu_sc as plsc

BATCH = 16
HIDDEN = 768
NUM_NODES = 26
MAXP = 8
LANES = 16
NCHUNK = HIDDEN // LANES
LN2 = 0.6931471805599453
SQRT2 = 1.4142135623730951

_PATHS = [
    ([0, 2, 3], [-1, 1, 1]),
    ([0, 2, 3, 5, 6], [-1, 1, -1, 1, 1]),
    ([0, 2, 14, 15, 17, 18], [-1, -1, 1, -1, 1, 1]),
    ([0, 2, 14, 20, 21], [-1, -1, -1, 1, 1]),
    ([0, 2, 3, 5, 8, 9], [-1, 1, -1, -1, 1, 1]),
    ([0, 2, 14, 15], [-1, -1, 1, 1]),
    ([0, 2, 14, 20, 23, 24], [-1, -1, -1, -1, 1, 1]),
    ([0, 2, 3, 5, 8, 11, 12], [-1, 1, -1, -1, -1, 1, 1]),
]

_IDX = np.zeros((BATCH, MAXP), dtype=np.int32)
_SGN = np.zeros((BATCH, LANES), dtype=np.float32)
_WGT = np.zeros((BATCH, LANES), dtype=np.float32)
for _b in range(BATCH):
    _idxs, _signs = _PATHS[_b % len(_PATHS)]
    for _j, (_i, _s) in enumerate(zip(_idxs, _signs)):
        _IDX[_b, _j] = _i
        _SGN[_b, _j] = float(_s)
        _WGT[_b, _j] = -1.0 / (BATCH * len(_idxs))

_MESH = plsc.VectorSubcoreMesh(
    core_axis_name="c", subcore_axis_name="s", num_cores=2, num_subcores=16)


_GATHER_DNUMS = lax.GatherDimensionNumbers(
    offset_dims=(), collapsed_slice_dims=(0,), start_index_map=(0,))


def _permute(x, idx):
    return lax.gather(x, idx[:, None], _GATHER_DNUMS, slice_sizes=(1,),
                      mode=lax.GatherScatterMode.PROMISE_IN_BOUNDS)


def _lane_total(x, lane):
    for sh in (8, 4, 2, 1):
        x = x + _permute(x, lane ^ sh)
    return x


def _log_apx(y):
    m = y
    e = jnp.zeros((LANES,), jnp.float32)
    for k in (16, 8, 4, 2, 1):
        scale = float(2 ** k)
        cond = m * scale < 2.0
        m = jnp.where(cond, m * scale, m)
        e = jnp.where(cond, e - float(k), e)
    big = m >= SQRT2
    m = jnp.where(big, m * 0.5, m)
    e = jnp.where(big, e + 1.0, e)
    t = (m - 1.0) / (m + 1.0)
    t2 = t * t
    poly = 1.0 + t2 * (1.0 / 3.0 + t2 * (1.0 / 5.0 + t2 * (1.0 / 7.0 + t2 * (1.0 / 9.0))))
    return e * LN2 + 2.0 * t * poly


@functools.partial(
    pl.kernel,
    out_type=(jax.ShapeDtypeStruct((LANES,), jnp.float32),
              jax.ShapeDtypeStruct((BATCH, LANES), jnp.float32)),
    mesh=_MESH,
    scratch_types=[
        pltpu.VMEM((MAXP,), jnp.int32),
        pltpu.VMEM((HIDDEN,), jnp.float32),
        pltpu.VMEM((MAXP, HIDDEN), jnp.float32),
        pltpu.VMEM((LANES,), jnp.float32),
        pltpu.VMEM((LANES,), jnp.float32),
        pltpu.VMEM((LANES,), jnp.float32),
        pltpu.VMEM((BATCH, LANES), jnp.float32),
        pltpu.SemaphoreType.DMA,
    ],
)
def _sc_kernel(outputs_hbm, table_hbm, idx_hbm, sgn_hbm, wgt_hbm,
               out_hbm, stage_hbm,
               idx_v, h_v, rows_v, sgn_v, wgt_v, loss_v, gath_v, sem):
    c = lax.axis_index("c")
    s = lax.axis_index("s")
    lane = jnp.arange(LANES, dtype=jnp.int32)

    @pl.when(c == 0)
    def _compute():
        pltpu.sync_copy(idx_hbm.at[s], idx_v)
        pltpu.sync_copy(outputs_hbm.at[s, 0], h_v)
        pltpu.sync_copy(sgn_hbm.at[s], sgn_v)
        pltpu.sync_copy(wgt_hbm.at[s], wgt_v)
        pltpu.async_copy(table_hbm.at[idx_v], rows_v, sem).wait()

        def chunk_body(j, accs):
            h = h_v[pl.ds(j * LANES, LANES)]
            return tuple(
                acc + rows_v[n, pl.ds(j * LANES, LANES)] * h
                for n, acc in enumerate(accs))

        accs = lax.fori_loop(
            0, NCHUNK, chunk_body,
            tuple(jnp.zeros((LANES,), jnp.float32) for _ in range(MAXP)))

        scores = jnp.zeros((LANES,), jnp.float32)
        for n in range(MAXP):
            scores = jnp.where(lane == n, _lane_total(accs[n], lane), scores)

        z = sgn_v[...] * scores
        p = 1.0 / (1.0 + jnp.exp(-z))
        loss_v[...] = wgt_v[...] * _log_apx(p + 1e-7)
        pltpu.sync_copy(loss_v, stage_hbm.at[s])

    plsc.subcore_barrier()

    @pl.when((c == 0) & (s == 0))
    def _reduce():
        pltpu.sync_copy(stage_hbm, gath_v)
        acc = jnp.zeros((LANES,), jnp.float32)
        for b in range(BATCH):
            acc = acc + gath_v[b]
        loss_v[...] = _lane_total(acc, lane)
        pltpu.sync_copy(loss_v, out_hbm)


@jax.jit
def kernel(outputs, table):
    out, _ = _sc_kernel(outputs, table, jnp.asarray(_IDX), jnp.asarray(_SGN),
                        jnp.asarray(_WGT))
    return out[0]


# speedup vs baseline: 1.1116x; 1.1116x over previous
import functools
import numpy as np
import jax
import jax.numpy as jnp
from jax import lax
from jax.experimental import pallas as pl
from jax.experimental.pallas import tpu as pltpu
from jax.experimental.pallas import tpu_sc as plsc

BATCH = 16
HIDDEN = 768
NUM_NODES = 26
MAXP = 8
LANES = 16
NCHUNK = HIDDEN // LANES
LN2 = 0.6931471805599453
SQRT2 = 1.4142135623730951

_PATHS = [
    ([0, 2, 3], [-1, 1, 1]),
    ([0, 2, 3, 5, 6], [-1, 1, -1, 1, 1]),
    ([0, 2, 14, 15, 17, 18], [-1, -1, 1, -1, 1, 1]),
    ([0, 2, 14, 20, 21], [-1, -1, -1, 1, 1]),
    ([0, 2, 3, 5, 8, 9], [-1, 1, -1, -1, 1, 1]),
    ([0, 2, 14, 15], [-1, -1, 1, 1]),
    ([0, 2, 14, 20, 23, 24], [-1, -1, -1, -1, 1, 1]),
    ([0, 2, 3, 5, 8, 11, 12], [-1, 1, -1, -1, -1, 1, 1]),
]

_IDX = np.zeros((BATCH, MAXP), dtype=np.int32)
_SW = np.zeros((BATCH, 2 * LANES), dtype=np.float32)
for _b in range(BATCH):
    _idxs, _signs = _PATHS[_b % len(_PATHS)]
    for _j, (_i, _s) in enumerate(zip(_idxs, _signs)):
        _IDX[_b, _j] = _i
        _SW[_b, _j] = float(_s)
        _SW[_b, LANES + _j] = -1.0 / (BATCH * len(_idxs))

_MESH = plsc.VectorSubcoreMesh(
    core_axis_name="c", subcore_axis_name="s", num_cores=2, num_subcores=16)


_GATHER_DNUMS = lax.GatherDimensionNumbers(
    offset_dims=(), collapsed_slice_dims=(0,), start_index_map=(0,))


def _permute(x, idx):
    return lax.gather(x, idx[:, None], _GATHER_DNUMS, slice_sizes=(1,),
                      mode=lax.GatherScatterMode.PROMISE_IN_BOUNDS)


def _lane_total(x, lane):
    for sh in (8, 4, 2, 1):
        x = x + _permute(x, lane ^ sh)
    return x


def _log_apx(y):
    m = y
    e = jnp.zeros((LANES,), jnp.float32)
    for k in (16, 8, 4, 2, 1):
        scale = float(2 ** k)
        cond = m * scale < 2.0
        m = jnp.where(cond, m * scale, m)
        e = jnp.where(cond, e - float(k), e)
    big = m >= SQRT2
    m = jnp.where(big, m * 0.5, m)
    e = jnp.where(big, e + 1.0, e)
    t = (m - 1.0) / (m + 1.0)
    t2 = t * t
    poly = 1.0 + t2 * (1.0 / 3.0 + t2 * (1.0 / 5.0 + t2 * (1.0 / 7.0 + t2 * (1.0 / 9.0))))
    return e * LN2 + 2.0 * t * poly


@functools.partial(
    pl.kernel,
    out_type=(jax.ShapeDtypeStruct((LANES,), jnp.float32),
              jax.ShapeDtypeStruct((BATCH, LANES), jnp.float32)),
    mesh=_MESH,
    scratch_types=[
        pltpu.VMEM((MAXP,), jnp.int32),
        pltpu.VMEM((HIDDEN,), jnp.float32),
        pltpu.VMEM((MAXP, HIDDEN), jnp.float32),
        pltpu.VMEM((2 * LANES,), jnp.float32),
        pltpu.VMEM((LANES,), jnp.float32),
        pltpu.VMEM((BATCH, LANES), jnp.float32),
        pltpu.SemaphoreType.DMA,
        pltpu.SemaphoreType.DMA,
        pltpu.SemaphoreType.DMA,
        pltpu.SemaphoreType.DMA,
    ],
)
def _sc_kernel(outputs_hbm, table_hbm, idx_hbm, sw_hbm,
               out_hbm, stage_hbm,
               idx_v, h_v, rows_v, sw_v, loss_v, gath_v,
               sem0, sem1, sem2, sem3):
    c = lax.axis_index("c")
    s = lax.axis_index("s")
    lane = jnp.arange(LANES, dtype=jnp.int32)

    @pl.when(c == 0)
    def _compute():
        cp_idx = pltpu.async_copy(idx_hbm.at[s], idx_v, sem0)
        cp_h = pltpu.async_copy(outputs_hbm.at[s, 0], h_v, sem1)
        cp_sw = pltpu.async_copy(sw_hbm.at[s], sw_v, sem2)
        cp_idx.wait()
        cp_rows = pltpu.async_copy(table_hbm.at[idx_v], rows_v, sem3)
        cp_h.wait()
        cp_rows.wait()
        cp_sw.wait()

        def chunk_body(j, accs):
            h = h_v[pl.ds(j * LANES, LANES)]
            return tuple(
                acc + rows_v[n, pl.ds(j * LANES, LANES)] * h
                for n, acc in enumerate(accs))

        accs = lax.fori_loop(
            0, NCHUNK, chunk_body,
            tuple(jnp.zeros((LANES,), jnp.float32) for _ in range(MAXP)))

        scores = jnp.zeros((LANES,), jnp.float32)
        for n in range(MAXP):
            scores = jnp.where(lane == n, _lane_total(accs[n], lane), scores)

        z = sw_v[pl.ds(0, LANES)] * scores
        p = 1.0 / (1.0 + jnp.exp(-z))
        loss_v[...] = sw_v[pl.ds(LANES, LANES)] * _log_apx(p + 1e-7)
        pltpu.sync_copy(loss_v, stage_hbm.at[s])

    plsc.subcore_barrier()

    @pl.when((c == 0) & (s == 0))
    def _reduce():
        pltpu.sync_copy(stage_hbm, gath_v)
        acc = jnp.zeros((LANES,), jnp.float32)
        for b in range(BATCH):
            acc = acc + gath_v[b]
        loss_v[...] = _lane_total(acc, lane)
        pltpu.sync_copy(loss_v, out_hbm)


@jax.jit
def kernel(outputs, table):
    out, _ = _sc_kernel(outputs, table, jnp.asarray(_IDX), jnp.asarray(_SW))
    return out[0]


# speedup vs baseline: 1.1785x; 1.0602x over previous
import functools
import numpy as np
import jax
import jax.numpy as jnp
from jax import lax
from jax.experimental import pallas as pl
from jax.experimental.pallas import tpu as pltpu
from jax.experimental.pallas import tpu_sc as plsc

BATCH = 16
HIDDEN = 768
NUM_NODES = 26
MAXP = 8
LANES = 16
NCHUNK = HIDDEN // LANES
LN2 = 0.6931471805599453
SQRT2 = 1.4142135623730951

_PATHS = [
    ([0, 2, 3], [-1, 1, 1]),
    ([0, 2, 3, 5, 6], [-1, 1, -1, 1, 1]),
    ([0, 2, 14, 15, 17, 18], [-1, -1, 1, -1, 1, 1]),
    ([0, 2, 14, 20, 21], [-1, -1, -1, 1, 1]),
    ([0, 2, 3, 5, 8, 9], [-1, 1, -1, -1, 1, 1]),
    ([0, 2, 14, 15], [-1, -1, 1, 1]),
    ([0, 2, 14, 20, 23, 24], [-1, -1, -1, -1, 1, 1]),
    ([0, 2, 3, 5, 8, 11, 12], [-1, 1, -1, -1, -1, 1, 1]),
]

_IDX = np.zeros((BATCH, MAXP), dtype=np.int32)
_SW = np.zeros((BATCH, 2 * LANES), dtype=np.float32)
for _b in range(BATCH):
    _idxs, _signs = _PATHS[_b % len(_PATHS)]
    for _j, (_i, _s) in enumerate(zip(_idxs, _signs)):
        _IDX[_b, _j] = _i
        _SW[_b, _j] = float(_s)
        _SW[_b, LANES + _j] = -1.0 / (BATCH * len(_idxs))

_MESH = plsc.VectorSubcoreMesh(
    core_axis_name="c", subcore_axis_name="s", num_cores=1, num_subcores=16)


_GATHER_DNUMS = lax.GatherDimensionNumbers(
    offset_dims=(), collapsed_slice_dims=(0,), start_index_map=(0,))


def _permute(x, idx):
    return lax.gather(x, idx[:, None], _GATHER_DNUMS, slice_sizes=(1,),
                      mode=lax.GatherScatterMode.PROMISE_IN_BOUNDS)


def _lane_total(x, lane):
    for sh in (8, 4, 2, 1):
        x = x + _permute(x, lane ^ sh)
    return x


def _log_apx(y):
    m = y
    e = jnp.zeros((LANES,), jnp.float32)
    for k in (16, 8, 4, 2, 1):
        scale = float(2 ** k)
        cond = m * scale < 2.0
        m = jnp.where(cond, m * scale, m)
        e = jnp.where(cond, e - float(k), e)
    big = m >= SQRT2
    m = jnp.where(big, m * 0.5, m)
    e = jnp.where(big, e + 1.0, e)
    t = (m - 1.0) / (m + 1.0)
    t2 = t * t
    poly = 1.0 + t2 * (1.0 / 3.0 + t2 * (1.0 / 5.0 + t2 * (1.0 / 7.0 + t2 * (1.0 / 9.0))))
    return e * LN2 + 2.0 * t * poly


@functools.partial(
    pl.kernel,
    out_type=(jax.ShapeDtypeStruct((LANES,), jnp.float32),
              jax.ShapeDtypeStruct((BATCH, LANES), jnp.float32)),
    mesh=_MESH,
    scratch_types=[
        pltpu.VMEM((MAXP,), jnp.int32),
        pltpu.VMEM((HIDDEN,), jnp.float32),
        pltpu.VMEM((MAXP, HIDDEN), jnp.float32),
        pltpu.VMEM((2 * LANES,), jnp.float32),
        pltpu.VMEM((LANES,), jnp.float32),
        pltpu.VMEM((BATCH, LANES), jnp.float32),
        pltpu.SemaphoreType.DMA,
        pltpu.SemaphoreType.DMA,
        pltpu.SemaphoreType.DMA,
        pltpu.SemaphoreType.DMA,
    ],
)
def _sc_kernel(outputs_hbm, table_hbm, idx_hbm, sw_hbm,
               out_hbm, stage_hbm,
               idx_v, h_v, rows_v, sw_v, loss_v, gath_v,
               sem0, sem1, sem2, sem3):
    c = lax.axis_index("c")
    s = lax.axis_index("s")
    lane = jnp.arange(LANES, dtype=jnp.int32)

    @pl.when(c == 0)
    def _compute():
        cp_idx = pltpu.async_copy(idx_hbm.at[s], idx_v, sem0)
        cp_h = pltpu.async_copy(outputs_hbm.at[s, 0], h_v, sem1)
        cp_sw = pltpu.async_copy(sw_hbm.at[s], sw_v, sem2)
        cp_idx.wait()
        cp_rows = pltpu.async_copy(table_hbm.at[idx_v], rows_v, sem3)
        cp_h.wait()
        cp_rows.wait()
        cp_sw.wait()

        def chunk_body(j, accs):
            h = h_v[pl.ds(j * LANES, LANES)]
            return tuple(
                acc + rows_v[n, pl.ds(j * LANES, LANES)] * h
                for n, acc in enumerate(accs))

        accs = lax.fori_loop(
            0, NCHUNK, chunk_body,
            tuple(jnp.zeros((LANES,), jnp.float32) for _ in range(MAXP)))

        scores = jnp.zeros((LANES,), jnp.float32)
        for n in range(MAXP):
            scores = jnp.where(lane == n, _lane_total(accs[n], lane), scores)

        z = sw_v[pl.ds(0, LANES)] * scores
        p = 1.0 / (1.0 + jnp.exp(-z))
        loss_v[...] = sw_v[pl.ds(LANES, LANES)] * _log_apx(p + 1e-7)
        pltpu.sync_copy(loss_v, stage_hbm.at[s])

    plsc.subcore_barrier()

    @pl.when((c == 0) & (s == 0))
    def _reduce():
        pltpu.sync_copy(stage_hbm, gath_v)
        acc = jnp.zeros((LANES,), jnp.float32)
        for b in range(BATCH):
            acc = acc + gath_v[b]
        loss_v[...] = _lane_total(acc, lane)
        pltpu.sync_copy(loss_v, out_hbm)


@jax.jit
def kernel(outputs, table):
    out, _ = _sc_kernel(outputs, table, jnp.asarray(_IDX), jnp.asarray(_SW))
    return out[0]
